# baseline (speedup 1.0000x reference)
import functools

import jax
import jax.numpy as jnp
from jax import lax
from jax.experimental import pallas as pl
from jax.experimental.pallas import tpu as pltpu
from jax.experimental.pallas import tpu_sc as plsc

D = 64
SUB = 128
TOTAL = 4096 * 200
NW = 32
ROWS_PER_W = TOTAL // NW // SUB
GPC = 4
CHUNK = GPC * SUB
NCHUNK = ROWS_PER_W // GPC

_mesh = plsc.VectorSubcoreMesh(core_axis_name="c", subcore_axis_name="s")


@functools.partial(
    pl.kernel,
    mesh=_mesh,
    out_type=jax.ShapeDtypeStruct((TOTAL, D), jnp.float32),
    compiler_params=pltpu.CompilerParams(use_tc_tiling_on_sc=False),
    scratch_types=[
        pltpu.VMEM((ROWS_PER_W, SUB), jnp.int32),
        pltpu.VMEM((D,), jnp.float32),
        pltpu.VMEM((CHUNK, D), jnp.float32),
        pltpu.VMEM((CHUNK, D), jnp.float32),
        pltpu.SemaphoreType.DMA,
        pltpu.SemaphoreType.DMA,
        pltpu.SemaphoreType.DMA,
        pltpu.SemaphoreType.DMA,
    ],
)
def _emb_kernel(idx_hbm, w_hbm, sh_hbm, out_hbm,
                idx_v, sh_v, buf_a, buf_b,
                gsem_a, gsem_b, osem_a, osem_b):
    wid = lax.axis_index("s") * 2 + lax.axis_index("c")
    row0 = wid * ROWS_PER_W

    pltpu.sync_copy(idx_hbm.at[pl.ds(row0, ROWS_PER_W)], idx_v)
    pltpu.sync_copy(sh_hbm, sh_v)

    bufs = (buf_a, buf_b)
    gsems = (gsem_a, gsem_b)
    osems = (osem_a, osem_b)

    def fire_gathers(chunk, buf, gsem):
        for q in range(GPC):
            pltpu.async_copy(
                w_hbm.at[idx_v.at[chunk * GPC + q]],
                buf.at[pl.ds(q * SUB, SUB)],
                gsem,
            )

    def drain_gathers(buf, gsem):
        for q in range(GPC):
            pltpu.make_async_copy(
                w_hbm.at[idx_v.at[0]],
                buf.at[pl.ds(q * SUB, SUB)],
                gsem,
            ).wait()

    sh0 = sh_v[pl.ds(0, 16)]
    sh1 = sh_v[pl.ds(16, 16)]
    sh2 = sh_v[pl.ds(32, 16)]
    sh3 = sh_v[pl.ds(48, 16)]

    def add_shared(buf):
        def body(i, _):
            buf[i, pl.ds(0, 16)] += sh0
            buf[i, pl.ds(16, 16)] += sh1
            buf[i, pl.ds(32, 16)] += sh2
            buf[i, pl.ds(48, 16)] += sh3
            return _
        lax.fori_loop(0, CHUNK, body, 0, unroll=4)

    fire_gathers(0, buf_a, gsem_a)

    out_base = wid * (ROWS_PER_W * SUB)

    def step(t, carry):
        for par in range(2):
            @pl.when((t % 2) == par)
            def _par_body(par=par):
                cur = bufs[par]
                nxt = bufs[1 - par]

                @pl.when(t >= 1)
                def _drain_prev_out():
                    pltpu.make_async_copy(
                        nxt, out_hbm.at[pl.ds(0, CHUNK)], osems[1 - par]
                    ).wait()

                @pl.when(t + 1 < NCHUNK)
                def _fire_next():
                    fire_gathers(t + 1, nxt, gsems[1 - par])

                drain_gathers(cur, gsems[par])
                add_shared(cur)
                pltpu.async_copy(
                    cur,
                    out_hbm.at[pl.ds(out_base + t * CHUNK, CHUNK)],
                    osems[par],
                )
        return carry

    lax.fori_loop(0, NCHUNK, step, 0, unroll=1)

    last = (NCHUNK - 1) % 2
    pltpu.make_async_copy(
        bufs[last], out_hbm.at[pl.ds(0, CHUNK)], osems[last]
    ).wait()


def kernel(x, embed_weight, shared_embed):
    idx = x.reshape(TOTAL // SUB, SUB).astype(jnp.int32)
    sh = shared_embed.reshape(D)
    out = _emb_kernel(idx, embed_weight, sh)
    return out.reshape(4096, 1, 200, 64)


# speedup vs baseline: 1.2188x; 1.2188x over previous
import functools

import jax
import jax.numpy as jnp
from jax import lax
from jax.experimental import pallas as pl
from jax.experimental.pallas import tpu as pltpu
from jax.experimental.pallas import tpu_sc as plsc

D = 64
DP = 128
SUB = 128
TOTAL = 4096 * 200
NW = 32
ROWS_PER_W = TOTAL // NW // SUB
GPC = 2
CHUNK = GPC * SUB
NCHUNK = ROWS_PER_W // GPC

_mesh = plsc.VectorSubcoreMesh(core_axis_name="c", subcore_axis_name="s")


@functools.partial(
    pl.kernel,
    mesh=_mesh,
    out_type=jax.ShapeDtypeStruct((TOTAL, DP), jnp.float32),
    scratch_types=[
        pltpu.VMEM((ROWS_PER_W, SUB), jnp.int32),
        pltpu.VMEM((D,), jnp.float32),
        pltpu.VMEM((CHUNK, DP), jnp.float32),
        pltpu.VMEM((CHUNK, DP), jnp.float32),
        pltpu.SemaphoreType.DMA,
        pltpu.SemaphoreType.DMA,
        pltpu.SemaphoreType.DMA,
        pltpu.SemaphoreType.DMA,
    ],
)
def _emb_kernel(idx_hbm, w_hbm, sh_hbm, out_hbm,
                idx_v, sh_v, buf_a, buf_b,
                gsem_a, gsem_b, osem_a, osem_b):
    wid = lax.axis_index("s") * 2 + lax.axis_index("c")
    row0 = wid * ROWS_PER_W

    pltpu.sync_copy(idx_hbm.at[pl.ds(row0, ROWS_PER_W)], idx_v)
    pltpu.sync_copy(sh_hbm, sh_v)

    bufs = (buf_a, buf_b)
    gsems = (gsem_a, gsem_b)
    osems = (osem_a, osem_b)

    def fire_gathers(chunk, buf, gsem):
        for q in range(GPC):
            pltpu.async_copy(
                w_hbm.at[idx_v.at[chunk * GPC + q]],
                buf.at[pl.ds(q * SUB, SUB)],
                gsem,
            )

    def drain_gathers(buf, gsem):
        for q in range(GPC):
            pltpu.make_async_copy(
                w_hbm.at[idx_v.at[0]],
                buf.at[pl.ds(q * SUB, SUB)],
                gsem,
            ).wait()

    sh0 = sh_v[pl.ds(0, 16)]
    sh1 = sh_v[pl.ds(16, 16)]
    sh2 = sh_v[pl.ds(32, 16)]
    sh3 = sh_v[pl.ds(48, 16)]

    def add_shared(buf):
        def body(i, carry):
            buf[i, pl.ds(0, 16)] += sh0
            buf[i, pl.ds(16, 16)] += sh1
            buf[i, pl.ds(32, 16)] += sh2
            buf[i, pl.ds(48, 16)] += sh3
            return carry
        lax.fori_loop(0, CHUNK, body, 0, unroll=4)

    fire_gathers(0, buf_a, gsem_a)

    out_base = wid * (ROWS_PER_W * SUB)

    def step(t, carry):
        for par in range(2):
            @pl.when((t % 2) == par)
            def _par_body(par=par):
                cur = bufs[par]
                nxt = bufs[1 - par]

                @pl.when(t >= 1)
                def _drain_prev_out():
                    pltpu.make_async_copy(
                        nxt, out_hbm.at[pl.ds(0, CHUNK)], osems[1 - par]
                    ).wait()

                @pl.when(t + 1 < NCHUNK)
                def _fire_next():
                    fire_gathers(t + 1, nxt, gsems[1 - par])

                drain_gathers(cur, gsems[par])
                add_shared(cur)
                pltpu.async_copy(
                    cur,
                    out_hbm.at[pl.ds(out_base + t * CHUNK, CHUNK)],
                    osems[par],
                )
        return carry

    lax.fori_loop(0, NCHUNK, step, 0, unroll=1)

    last = (NCHUNK - 1) % 2
    pltpu.make_async_copy(
        bufs[last], out_hbm.at[pl.ds(0, CHUNK)], osems[last]
    ).wait()


def kernel(x, embed_weight, shared_embed):
    idx = x.reshape(TOTAL // SUB, SUB).astype(jnp.int32)
    w128 = jnp.pad(embed_weight, ((0, 0), (0, DP - D)))
    sh = shared_embed.reshape(D)
    out = _emb_kernel(idx, w128, sh)
    return out[:, :D].reshape(4096, 1, 200, 64)
